# baseline (speedup 1.0000x reference)
import functools

import jax
import jax.numpy as jnp
from jax import lax
from jax.experimental import pallas as pl
from jax.experimental.pallas import tpu as pltpu
from jax.experimental.pallas import tpu_sc as plsc

NUM_CORES = 2
NUM_SUBCORES = 16
NUM_WORKERS = NUM_CORES * NUM_SUBCORES
IDX_PER_STREAM = 128
STREAMS_PER_CHUNK = 8
CHUNK = IDX_PER_STREAM * STREAMS_PER_CHUNK
GROUP = 8
HALF = 4
LANES = 16


def _node_proj_kernel(na_ref, w1s_ref, w1r_ref, ps_ref, pr_ref):
    na = na_ref[...]
    ps_ref[...] = jnp.dot(na, w1s_ref[...], preferred_element_type=jnp.float32)
    pr_ref[...] = jnp.dot(na, w1r_ref[...], preferred_element_type=jnp.float32)


def _make_sc_gather(e_pad, latent, chunks_c0, chunks_c1):
    rows_per_chunk = STREAMS_PER_CHUNK
    groups_per_chunk = CHUNK // GROUP
    packed_width = HALF * latent
    chunks_pair = chunks_c0 + chunks_c1
    mesh = plsc.VectorSubcoreMesh(core_axis_name="c", subcore_axis_name="s")

    @functools.partial(
        pl.kernel,
        out_type=(
            jax.ShapeDtypeStruct((e_pad // GROUP, packed_width), jnp.float32),
            jax.ShapeDtypeStruct((e_pad // GROUP, packed_width), jnp.float32),
        ),
        mesh=mesh,
        compiler_params=pltpu.CompilerParams(use_tc_tiling_on_sc=False),
        scratch_types=[
            pltpu.VMEM((rows_per_chunk, IDX_PER_STREAM), jnp.int32),
            pltpu.VMEM((rows_per_chunk, IDX_PER_STREAM), jnp.int32),
            pltpu.VMEM((CHUNK, latent), jnp.float32),
            pltpu.VMEM((CHUNK, latent), jnp.float32),
            pltpu.VMEM((groups_per_chunk, packed_width), jnp.float32),
            pltpu.VMEM((groups_per_chunk, packed_width), jnp.float32),
            pltpu.SemaphoreType.DMA,
            pltpu.SemaphoreType.DMA,
        ],
    )
    def sc_gather(sidx_hbm, ridx_hbm, ps_hbm, pr_hbm, ta_hbm, tb_hbm,
                  idxs_v, idxr_v, bufs_v, bufr_v, bufta_v, buftb_v, sem_s, sem_r):
        c = lax.axis_index("c")
        s = lax.axis_index("s")
        chunk_base = s * chunks_pair + c * chunks_c0
        n_chunks = jnp.where(c == 0, chunks_c0, chunks_c1)

        def chunk_body(g, carry):
            gc = chunk_base + g
            idx_row = gc * rows_per_chunk
            pltpu.sync_copy(sidx_hbm.at[pl.ds(idx_row, rows_per_chunk)], idxs_v)
            pltpu.sync_copy(ridx_hbm.at[pl.ds(idx_row, rows_per_chunk)], idxr_v)
            copies = []
            for j in range(STREAMS_PER_CHUNK):
                dst = pl.ds(j * IDX_PER_STREAM, IDX_PER_STREAM)
                copies.append(
                    pltpu.async_copy(ps_hbm.at[idxs_v.at[j]], bufs_v.at[dst], sem_s))
                copies.append(
                    pltpu.async_copy(pr_hbm.at[idxr_v.at[j]], bufr_v.at[dst], sem_r))
            for cp in copies:
                cp.wait()

            def pack_body(q, c2):
                i0 = q * GROUP
                for u in range(GROUP):
                    i = i0 + u
                    buf = bufta_v if u < HALF else buftb_v
                    col0 = (u % HALF) * latent
                    for half in range(latent // LANES):
                        col = half * LANES
                        a = bufs_v[i, pl.ds(col, LANES)] + bufr_v[i, pl.ds(col, LANES)]
                        buf[q, pl.ds(col0 + col, LANES)] = a
                return c2

            lax.fori_loop(0, groups_per_chunk, pack_body, 0)
            out_base = gc * groups_per_chunk
            pltpu.sync_copy(bufta_v, ta_hbm.at[pl.ds(out_base, groups_per_chunk)])
            pltpu.sync_copy(buftb_v, tb_hbm.at[pl.ds(out_base, groups_per_chunk)])
            return carry

        lax.fori_loop(0, n_chunks, chunk_body, 0)

    return sc_gather


def _edge_mlp_kernel(ea_ref, ta_ref, tb_ref, ga_ref, w1e_ref, w1g_ref, b1_ref,
                     w2_ref, b2_ref, out_ref):
    latent = w1g_ref.shape[1]
    pw = HALF * latent
    gvec = jnp.dot(ga_ref[...], w1g_ref[...], preferred_element_type=jnp.float32)
    gvec = gvec + b1_ref[...]
    gvec = jnp.concatenate([gvec] * HALF, axis=1)
    eaw = jnp.dot(ea_ref[...], w1e_ref[...], preferred_element_type=jnp.float32)
    ha = jnp.maximum(ta_ref[...] + eaw[:, :pw] + gvec, 0.0)
    hb = jnp.maximum(tb_ref[...] + eaw[:, pw:] + gvec, 0.0)
    outs = []
    for w in range(GROUP):
        h = ha if w < HALF else hb
        h_w = h[:, (w % HALF) * latent:(w % HALF + 1) * latent]
        outs.append(
            jnp.dot(h_w, w2_ref[...], preferred_element_type=jnp.float32) + b2_ref[...])
    out_ref[...] = jnp.concatenate(outs, axis=1)


def kernel(node_attr, edge_attr, global_attr, edge_index, ng_index, eg_index,
           W1, b1, W2, b2):
    n_nodes, d_feat = node_attr.shape
    n_edges, d_edge = edge_attr.shape
    d_global = global_attr.shape[1]
    latent = W1.shape[1]
    out_dim = W2.shape[1]

    w1_e = W1[:d_edge]
    w1_s = W1[d_edge:d_edge + d_feat]
    w1_r = W1[d_edge + d_feat:d_edge + 2 * d_feat]
    w1_g = W1[d_edge + 2 * d_feat:]

    proj_s, proj_r = pl.pallas_call(
        _node_proj_kernel,
        out_shape=(
            jax.ShapeDtypeStruct((n_nodes, latent), jnp.float32),
            jax.ShapeDtypeStruct((n_nodes, latent), jnp.float32),
        ),
    )(node_attr, w1_s, w1_r)

    per_worker_unit = NUM_WORKERS * CHUNK
    e_pad = ((n_edges + per_worker_unit - 1) // per_worker_unit) * per_worker_unit
    chunks_per_worker = e_pad // per_worker_unit
    chunks_c0 = max(1, (2 * chunks_per_worker * 13) // 20)
    chunks_c1 = 2 * chunks_per_worker - chunks_c0

    s_idx = edge_index[0].astype(jnp.int32)
    r_idx = edge_index[1].astype(jnp.int32)
    pad = e_pad - n_edges
    s_idx = jnp.pad(s_idx, (0, pad)).reshape(e_pad // IDX_PER_STREAM, IDX_PER_STREAM)
    r_idx = jnp.pad(r_idx, (0, pad)).reshape(e_pad // IDX_PER_STREAM, IDX_PER_STREAM)

    ta, tb = _make_sc_gather(e_pad, latent, chunks_c0, chunks_c1)(
        s_idx, r_idx, proj_s, proj_r)

    eb = 8000
    grid = n_edges // eb
    gb = eb // GROUP
    ea8 = edge_attr.reshape(n_edges // GROUP, GROUP * d_edge)
    w1e_bd = jnp.kron(jnp.eye(GROUP, dtype=jnp.float32), w1_e)
    out8 = pl.pallas_call(
        _edge_mlp_kernel,
        grid=(grid,),
        in_specs=[
            pl.BlockSpec((gb, GROUP * d_edge), lambda i: (i, 0)),
            pl.BlockSpec((gb, HALF * latent), lambda i: (i, 0)),
            pl.BlockSpec((gb, HALF * latent), lambda i: (i, 0)),
            pl.BlockSpec((1, d_global), lambda i: (0, 0)),
            pl.BlockSpec((GROUP * d_edge, GROUP * latent), lambda i: (0, 0)),
            pl.BlockSpec((d_global, latent), lambda i: (0, 0)),
            pl.BlockSpec((1, latent), lambda i: (0, 0)),
            pl.BlockSpec((latent, out_dim), lambda i: (0, 0)),
            pl.BlockSpec((1, out_dim), lambda i: (0, 0)),
        ],
        out_specs=pl.BlockSpec((gb, GROUP * out_dim), lambda i: (i, 0)),
        out_shape=jax.ShapeDtypeStruct((n_edges // GROUP, GROUP * out_dim), jnp.float32),
    )(ea8, ta, tb, global_attr, w1e_bd, w1_g,
      b1.reshape(1, latent), W2, b2.reshape(1, out_dim))
    return out8.reshape(n_edges, out_dim)


# speedup vs baseline: 1.2327x; 1.2327x over previous
import functools

import jax
import jax.numpy as jnp
import numpy as np
from jax import lax
from jax.experimental import pallas as pl
from jax.experimental.pallas import tpu as pltpu
from jax.experimental.pallas import tpu_sc as plsc

NUM_CORES = 2
NUM_SUBCORES = 16
NUM_WORKERS = NUM_CORES * NUM_SUBCORES
IDX_PER_STREAM = 128
STREAMS_PER_CHUNK = 8
CHUNK = IDX_PER_STREAM * STREAMS_PER_CHUNK
PACK = 4
LANES = 16


def _node_proj_kernel(na_ref, w1s_ref, w1r_ref, ps_ref, pr_ref):
    na = na_ref[...]
    ps_ref[...] = jnp.dot(
        na, w1s_ref[...], preferred_element_type=jnp.float32).astype(jnp.bfloat16)
    pr_ref[...] = jnp.dot(
        na, w1r_ref[...], preferred_element_type=jnp.float32).astype(jnp.bfloat16)


def _make_sc_gather(e_pad, latent, chunks_c0, chunks_c1):
    rows_per_chunk = STREAMS_PER_CHUNK
    packed_per_chunk = CHUNK // PACK
    packed_width = PACK * latent
    chunks_pair = chunks_c0 + chunks_c1
    mesh = plsc.VectorSubcoreMesh(core_axis_name="c", subcore_axis_name="s")

    @functools.partial(
        pl.kernel,
        out_type=jax.ShapeDtypeStruct((e_pad // PACK, packed_width), jnp.float32),
        mesh=mesh,
        compiler_params=pltpu.CompilerParams(
            use_tc_tiling_on_sc=False, needs_layout_passes=False),
        scratch_types=[
            pltpu.VMEM((rows_per_chunk, IDX_PER_STREAM), jnp.int32),
            pltpu.VMEM((rows_per_chunk, IDX_PER_STREAM), jnp.int32),
            pltpu.VMEM((CHUNK, latent), jnp.bfloat16),
            pltpu.VMEM((CHUNK, latent), jnp.bfloat16),
            pltpu.VMEM((packed_per_chunk, packed_width), jnp.float32),
            pltpu.SemaphoreType.DMA,
            pltpu.SemaphoreType.DMA,
        ],
    )
    def sc_gather(sidx_hbm, ridx_hbm, ps_hbm, pr_hbm, t_hbm,
                  idxs_v, idxr_v, bufs_v, bufr_v, buft_v, sem_s, sem_r):
        c = lax.axis_index("c")
        s = lax.axis_index("s")
        chunk_base = s * chunks_pair + c * chunks_c0
        n_chunks = jnp.where(c == 0, chunks_c0, chunks_c1)

        def chunk_body(g, carry):
            gc = chunk_base + g
            idx_row = gc * rows_per_chunk
            pltpu.sync_copy(sidx_hbm.at[pl.ds(idx_row, rows_per_chunk)], idxs_v)
            pltpu.sync_copy(ridx_hbm.at[pl.ds(idx_row, rows_per_chunk)], idxr_v)
            copies = []
            for j in range(STREAMS_PER_CHUNK):
                dst = pl.ds(j * IDX_PER_STREAM, IDX_PER_STREAM)
                copies.append(
                    pltpu.async_copy(ps_hbm.at[idxs_v.at[j]], bufs_v.at[dst], sem_s))
                copies.append(
                    pltpu.async_copy(pr_hbm.at[idxr_v.at[j]], bufr_v.at[dst], sem_r))
            for cp in copies:
                cp.wait()

            def pack_body(q, c2):
                i0 = q * PACK
                for u in range(PACK):
                    i = i0 + u
                    sa, sb = plsc.unpack(bufs_v[i, :],
                                         format=plsc.PackFormat.INTERLEAVED)
                    ra, rb = plsc.unpack(bufr_v[i, :],
                                         format=plsc.PackFormat.INTERLEAVED)
                    col0 = u * latent
                    buft_v[q, pl.ds(col0, LANES)] = sa + ra
                    buft_v[q, pl.ds(col0 + LANES, LANES)] = sb + rb
                return c2

            lax.fori_loop(0, packed_per_chunk, pack_body, 0)
            out_base = gc * packed_per_chunk
            pltpu.sync_copy(buft_v, t_hbm.at[pl.ds(out_base, packed_per_chunk)])
            return carry

        lax.fori_loop(0, n_chunks, chunk_body, 0)

    return sc_gather


def _edge_mlp_kernel(ea_ref, t_ref, ga_ref, w1e_ref, w1g_ref, b1_ref,
                     w2_ref, b2_ref, out_ref):
    pb, pw = t_ref.shape
    latent = pw // PACK
    gvec = jnp.dot(ga_ref[...], w1g_ref[...], preferred_element_type=jnp.float32)
    gvec = gvec + b1_ref[...]
    gvec = jnp.concatenate([gvec] * PACK, axis=1)
    pre = (t_ref[...]
           + jnp.dot(ea_ref[...], w1e_ref[...], preferred_element_type=jnp.float32)
           + gvec)
    h = jnp.maximum(pre, 0.0)
    for u in range(PACK):
        h_u = h[:, u * latent:(u + 1) * latent]
        o_u = jnp.dot(h_u, w2_ref[...], preferred_element_type=jnp.float32) + b2_ref[...]
        out_ref[pl.Slice(u, pb, PACK), :] = o_u


def kernel(node_attr, edge_attr, global_attr, edge_index, ng_index, eg_index,
           W1, b1, W2, b2):
    n_nodes, d_feat = node_attr.shape
    n_edges, d_edge = edge_attr.shape
    d_global = global_attr.shape[1]
    latent = W1.shape[1]
    out_dim = W2.shape[1]

    w1_e = W1[:d_edge]
    w1_s = W1[d_edge:d_edge + d_feat]
    w1_r = W1[d_edge + d_feat:d_edge + 2 * d_feat]
    w1_g = W1[d_edge + 2 * d_feat:]

    half = latent // 2
    perm = np.empty(latent, dtype=np.int32)
    perm[0::2] = np.arange(half)
    perm[1::2] = np.arange(half) + half

    proj_s, proj_r = pl.pallas_call(
        _node_proj_kernel,
        out_shape=(
            jax.ShapeDtypeStruct((n_nodes, latent), jnp.bfloat16),
            jax.ShapeDtypeStruct((n_nodes, latent), jnp.bfloat16),
        ),
    )(node_attr, w1_s[:, perm], w1_r[:, perm])

    per_worker_unit = NUM_WORKERS * CHUNK
    e_pad = ((n_edges + per_worker_unit - 1) // per_worker_unit) * per_worker_unit
    chunks_per_worker = e_pad // per_worker_unit
    chunks_c0 = max(1, (2 * chunks_per_worker * 13) // 20)
    chunks_c1 = 2 * chunks_per_worker - chunks_c0

    s_idx = edge_index[0].astype(jnp.int32)
    r_idx = edge_index[1].astype(jnp.int32)
    pad = e_pad - n_edges
    s_idx = jnp.pad(s_idx, (0, pad)).reshape(e_pad // IDX_PER_STREAM, IDX_PER_STREAM)
    r_idx = jnp.pad(r_idx, (0, pad)).reshape(e_pad // IDX_PER_STREAM, IDX_PER_STREAM)

    t_packed = _make_sc_gather(e_pad, latent, chunks_c0, chunks_c1)(
        s_idx, r_idx, proj_s, proj_r)

    eb = 4000
    grid = n_edges // eb
    pb = eb // PACK
    ea_packed = edge_attr.reshape(n_edges // PACK, PACK * d_edge)
    w1e_bd = jnp.kron(jnp.eye(PACK, dtype=jnp.float32), w1_e)
    out = pl.pallas_call(
        _edge_mlp_kernel,
        grid=(grid,),
        in_specs=[
            pl.BlockSpec((pb, PACK * d_edge), lambda i: (i, 0)),
            pl.BlockSpec((pb, PACK * latent), lambda i: (i, 0)),
            pl.BlockSpec((1, d_global), lambda i: (0, 0)),
            pl.BlockSpec((PACK * d_edge, PACK * latent), lambda i: (0, 0)),
            pl.BlockSpec((d_global, latent), lambda i: (0, 0)),
            pl.BlockSpec((1, latent), lambda i: (0, 0)),
            pl.BlockSpec((latent, out_dim), lambda i: (0, 0)),
            pl.BlockSpec((1, out_dim), lambda i: (0, 0)),
        ],
        out_specs=pl.BlockSpec((eb, out_dim), lambda i: (i, 0)),
        out_shape=jax.ShapeDtypeStruct((n_edges, out_dim), jnp.float32),
    )(ea_packed, t_packed, global_attr, w1e_bd, w1_g,
      b1.reshape(1, latent), W2, b2.reshape(1, out_dim))
    return out


# speedup vs baseline: 1.2779x; 1.0367x over previous
import functools

import jax
import jax.numpy as jnp
import numpy as np
from jax import lax
from jax.experimental import pallas as pl
from jax.experimental.pallas import tpu as pltpu
from jax.experimental.pallas import tpu_sc as plsc

NUM_CORES = 2
NUM_SUBCORES = 16
NUM_WORKERS = NUM_CORES * NUM_SUBCORES
IDX_PER_STREAM = 128
STREAMS_PER_CHUNK = 8
CHUNK = IDX_PER_STREAM * STREAMS_PER_CHUNK
PACK = 4
LANES = 16


def _node_proj_kernel(na_ref, w1s_ref, w1r_ref, ps_ref, pr_ref):
    na = na_ref[...]
    ps_ref[...] = jnp.dot(
        na, w1s_ref[...], preferred_element_type=jnp.float32).astype(jnp.bfloat16)
    pr_ref[...] = jnp.dot(
        na, w1r_ref[...], preferred_element_type=jnp.float32).astype(jnp.bfloat16)


def _make_sc_gather(e_pad, latent, chunks_per_worker):
    rows_per_chunk = STREAMS_PER_CHUNK
    packed_per_chunk = CHUNK // PACK
    packed_width = PACK * latent
    n_chunks = chunks_per_worker
    mesh = plsc.VectorSubcoreMesh(core_axis_name="c", subcore_axis_name="s")

    @functools.partial(
        pl.kernel,
        out_type=jax.ShapeDtypeStruct((e_pad // PACK, packed_width), jnp.float32),
        mesh=mesh,
        compiler_params=pltpu.CompilerParams(
            use_tc_tiling_on_sc=False, needs_layout_passes=False),
        scratch_types=[
            pltpu.VMEM((2, rows_per_chunk, IDX_PER_STREAM), jnp.int32),
            pltpu.VMEM((2, rows_per_chunk, IDX_PER_STREAM), jnp.int32),
            pltpu.VMEM((2 * CHUNK, latent), jnp.bfloat16),
            pltpu.VMEM((2 * CHUNK, latent), jnp.bfloat16),
            pltpu.VMEM((packed_per_chunk, packed_width), jnp.float32),
            pltpu.SemaphoreType.DMA,
            pltpu.SemaphoreType.DMA,
        ],
    )
    def sc_gather(sidx_hbm, ridx_hbm, ps_hbm, pr_hbm, t_hbm,
                  idxs_v, idxr_v, bufs_v, bufr_v, buft_v, sem0, sem1):
        c = lax.axis_index("c")
        s = lax.axis_index("s")
        wid = s * NUM_CORES + c
        chunk_base = wid * n_chunks
        sems = (sem0, sem1)

        def fire(g, b):
            idx_row = (chunk_base + g) * rows_per_chunk
            pltpu.sync_copy(sidx_hbm.at[pl.ds(idx_row, rows_per_chunk)],
                            idxs_v.at[b])
            pltpu.sync_copy(ridx_hbm.at[pl.ds(idx_row, rows_per_chunk)],
                            idxr_v.at[b])
            for j in range(STREAMS_PER_CHUNK):
                dst = pl.ds(b * CHUNK + j * IDX_PER_STREAM, IDX_PER_STREAM)
                pltpu.async_copy(ps_hbm.at[idxs_v.at[b, j]], bufs_v.at[dst], sems[b])
                pltpu.async_copy(pr_hbm.at[idxr_v.at[b, j]], bufr_v.at[dst], sems[b])

        def drain(b):
            for j in range(STREAMS_PER_CHUNK):
                dst = pl.ds(b * CHUNK + j * IDX_PER_STREAM, IDX_PER_STREAM)
                pltpu.make_async_copy(
                    ps_hbm.at[idxs_v.at[b, j]], bufs_v.at[dst], sems[b]).wait()
                pltpu.make_async_copy(
                    pr_hbm.at[idxr_v.at[b, j]], bufr_v.at[dst], sems[b]).wait()

        def pack_store(g, b):
            base = b * CHUNK

            def pack_body(q, c2):
                i0 = base + q * PACK
                for u in range(PACK):
                    i = i0 + u
                    sa, sb = plsc.unpack(bufs_v[i, :],
                                         format=plsc.PackFormat.INTERLEAVED)
                    ra, rb = plsc.unpack(bufr_v[i, :],
                                         format=plsc.PackFormat.INTERLEAVED)
                    col0 = u * latent
                    buft_v[q, pl.ds(col0, LANES)] = sa + ra
                    buft_v[q, pl.ds(col0 + LANES, LANES)] = sb + rb
                return c2

            lax.fori_loop(0, packed_per_chunk, pack_body, 0)
            out_base = (chunk_base + g) * packed_per_chunk
            pltpu.sync_copy(buft_v, t_hbm.at[pl.ds(out_base, packed_per_chunk)])

        fire(0, 0)

        def pair_body(p, carry):
            g0 = p * 2
            drain(0)
            fire(g0 + 1, 1)
            pack_store(g0, 0)
            drain(1)

            @pl.when(p < n_chunks // 2 - 1)
            def _():
                fire(g0 + 2, 0)

            pack_store(g0 + 1, 1)
            return carry

        lax.fori_loop(0, n_chunks // 2, pair_body, 0)

    return sc_gather


def _edge_mlp_kernel(ea_ref, t_ref, ga_ref, w1e_ref, w1g_ref, b1_ref,
                     w2_ref, b2_ref, out_ref):
    pb, pw = t_ref.shape
    latent = pw // PACK
    gvec = jnp.dot(ga_ref[...], w1g_ref[...], preferred_element_type=jnp.float32)
    gvec = gvec + b1_ref[...]
    gvec = jnp.concatenate([gvec] * PACK, axis=1)
    pre = (t_ref[...]
           + jnp.dot(ea_ref[...], w1e_ref[...], preferred_element_type=jnp.float32)
           + gvec)
    h = jnp.maximum(pre, 0.0)
    for u in range(PACK):
        h_u = h[:, u * latent:(u + 1) * latent]
        o_u = jnp.dot(h_u, w2_ref[...], preferred_element_type=jnp.float32) + b2_ref[...]
        out_ref[pl.Slice(u, pb, PACK), :] = o_u


def kernel(node_attr, edge_attr, global_attr, edge_index, ng_index, eg_index,
           W1, b1, W2, b2):
    n_nodes, d_feat = node_attr.shape
    n_edges, d_edge = edge_attr.shape
    d_global = global_attr.shape[1]
    latent = W1.shape[1]
    out_dim = W2.shape[1]

    w1_e = W1[:d_edge]
    w1_s = W1[d_edge:d_edge + d_feat]
    w1_r = W1[d_edge + d_feat:d_edge + 2 * d_feat]
    w1_g = W1[d_edge + 2 * d_feat:]

    half = latent // 2
    perm = np.empty(latent, dtype=np.int32)
    perm[0::2] = np.arange(half)
    perm[1::2] = np.arange(half) + half

    proj_s, proj_r = pl.pallas_call(
        _node_proj_kernel,
        out_shape=(
            jax.ShapeDtypeStruct((n_nodes, latent), jnp.bfloat16),
            jax.ShapeDtypeStruct((n_nodes, latent), jnp.bfloat16),
        ),
    )(node_attr, w1_s[:, perm], w1_r[:, perm])

    per_worker_unit = NUM_WORKERS * CHUNK
    e_pad = ((n_edges + per_worker_unit - 1) // per_worker_unit) * per_worker_unit
    chunks_per_worker = e_pad // per_worker_unit

    s_idx = edge_index[0].astype(jnp.int32)
    r_idx = edge_index[1].astype(jnp.int32)
    pad = e_pad - n_edges
    s_idx = jnp.pad(s_idx, (0, pad)).reshape(e_pad // IDX_PER_STREAM, IDX_PER_STREAM)
    r_idx = jnp.pad(r_idx, (0, pad)).reshape(e_pad // IDX_PER_STREAM, IDX_PER_STREAM)

    t_packed = _make_sc_gather(e_pad, latent, chunks_per_worker)(
        s_idx, r_idx, proj_s, proj_r)

    eb = 4000
    grid = n_edges // eb
    pb = eb // PACK
    ea_packed = edge_attr.reshape(n_edges // PACK, PACK * d_edge)
    w1e_bd = jnp.kron(jnp.eye(PACK, dtype=jnp.float32), w1_e)
    out = pl.pallas_call(
        _edge_mlp_kernel,
        grid=(grid,),
        in_specs=[
            pl.BlockSpec((pb, PACK * d_edge), lambda i: (i, 0)),
            pl.BlockSpec((pb, PACK * latent), lambda i: (i, 0)),
            pl.BlockSpec((1, d_global), lambda i: (0, 0)),
            pl.BlockSpec((PACK * d_edge, PACK * latent), lambda i: (0, 0)),
            pl.BlockSpec((d_global, latent), lambda i: (0, 0)),
            pl.BlockSpec((1, latent), lambda i: (0, 0)),
            pl.BlockSpec((latent, out_dim), lambda i: (0, 0)),
            pl.BlockSpec((1, out_dim), lambda i: (0, 0)),
        ],
        out_specs=pl.BlockSpec((eb, out_dim), lambda i: (i, 0)),
        out_shape=jax.ShapeDtypeStruct((n_edges, out_dim), jnp.float32),
    )(ea_packed, t_packed, global_attr, w1e_bd, w1_g,
      b1.reshape(1, latent), W2, b2.reshape(1, out_dim))
    return out


# speedup vs baseline: 1.3864x; 1.0849x over previous
import functools

import jax
import jax.numpy as jnp
import numpy as np
from jax import lax
from jax.experimental import pallas as pl
from jax.experimental.pallas import tpu as pltpu
from jax.experimental.pallas import tpu_sc as plsc

NUM_CORES = 2
NUM_SUBCORES = 16
NUM_WORKERS = NUM_CORES * NUM_SUBCORES
IDX_PER_STREAM = 128
STREAMS_PER_CHUNK = 8
CHUNK = IDX_PER_STREAM * STREAMS_PER_CHUNK
PACK = 4
LANES = 16


def _node_proj_kernel(na_ref, w1s_ref, w1r_ref, ps_ref, pr_ref):
    na = na_ref[...]
    ps_ref[...] = jnp.dot(
        na, w1s_ref[...], preferred_element_type=jnp.float32).astype(jnp.bfloat16)
    pr_ref[...] = jnp.dot(
        na, w1r_ref[...], preferred_element_type=jnp.float32).astype(jnp.bfloat16)


def _make_sc_gather(e_pad, latent, chunks_per_worker):
    rows_per_chunk = STREAMS_PER_CHUNK
    packed_per_chunk = CHUNK // PACK
    packed_width = PACK * latent
    n_chunks = chunks_per_worker
    mesh = plsc.VectorSubcoreMesh(core_axis_name="c", subcore_axis_name="s")

    @functools.partial(
        pl.kernel,
        out_type=jax.ShapeDtypeStruct((e_pad // PACK, packed_width), jnp.float32),
        mesh=mesh,
        compiler_params=pltpu.CompilerParams(
            use_tc_tiling_on_sc=False, needs_layout_passes=False),
        scratch_types=[
            pltpu.VMEM((2, rows_per_chunk, IDX_PER_STREAM), jnp.int32),
            pltpu.VMEM((2, rows_per_chunk, IDX_PER_STREAM), jnp.int32),
            pltpu.VMEM((2 * CHUNK, latent), jnp.bfloat16),
            pltpu.VMEM((2 * CHUNK, latent), jnp.bfloat16),
            pltpu.VMEM((packed_per_chunk, packed_width), jnp.float32),
            pltpu.SemaphoreType.DMA,
            pltpu.SemaphoreType.DMA,
        ],
    )
    def sc_gather(sidx_hbm, ridx_hbm, ps_hbm, pr_hbm, t_hbm,
                  idxs_v, idxr_v, bufs_v, bufr_v, buft_v, sem0, sem1):
        c = lax.axis_index("c")
        s = lax.axis_index("s")
        wid = s * NUM_CORES + c
        chunk_base = wid * n_chunks
        sems = (sem0, sem1)

        def fire(g, b):
            idx_row = (chunk_base + g) * rows_per_chunk
            pltpu.sync_copy(sidx_hbm.at[pl.ds(idx_row, rows_per_chunk)],
                            idxs_v.at[b])
            pltpu.sync_copy(ridx_hbm.at[pl.ds(idx_row, rows_per_chunk)],
                            idxr_v.at[b])
            for j in range(STREAMS_PER_CHUNK):
                dst = pl.ds(b * CHUNK + j * IDX_PER_STREAM, IDX_PER_STREAM)
                pltpu.async_copy(ps_hbm.at[idxs_v.at[b, j]], bufs_v.at[dst], sems[b])
                pltpu.async_copy(pr_hbm.at[idxr_v.at[b, j]], bufr_v.at[dst], sems[b])

        def drain(b):
            for j in range(STREAMS_PER_CHUNK):
                dst = pl.ds(b * CHUNK + j * IDX_PER_STREAM, IDX_PER_STREAM)
                pltpu.make_async_copy(
                    ps_hbm.at[idxs_v.at[b, j]], bufs_v.at[dst], sems[b]).wait()
                pltpu.make_async_copy(
                    pr_hbm.at[idxr_v.at[b, j]], bufr_v.at[dst], sems[b]).wait()

        def pack_store(g, b):
            base = b * CHUNK

            def pack_body(q, c2):
                i0 = base + q * PACK
                for u in range(PACK):
                    i = i0 + u
                    sa, sb = plsc.unpack(bufs_v[i, :],
                                         format=plsc.PackFormat.INTERLEAVED)
                    ra, rb = plsc.unpack(bufr_v[i, :],
                                         format=plsc.PackFormat.INTERLEAVED)
                    col0 = u * latent
                    buft_v[q, pl.ds(col0, LANES)] = sa + ra
                    buft_v[q, pl.ds(col0 + LANES, LANES)] = sb + rb
                return c2

            lax.fori_loop(0, packed_per_chunk, pack_body, 0)
            out_base = (chunk_base + g) * packed_per_chunk
            pltpu.sync_copy(buft_v, t_hbm.at[pl.ds(out_base, packed_per_chunk)])

        fire(0, 0)

        def pair_body(p, carry):
            g0 = p * 2
            drain(0)
            fire(g0 + 1, 1)
            pack_store(g0, 0)
            drain(1)

            @pl.when(p < n_chunks // 2 - 1)
            def _():
                fire(g0 + 2, 0)

            pack_store(g0 + 1, 1)
            return carry

        lax.fori_loop(0, n_chunks // 2, pair_body, 0)

    return sc_gather


def _edge_mlp_kernel(ea_ref, t_ref, ga_ref, w1e_ref, w1g_ref, b1_ref,
                     w2_ref, b2_ref, out_ref):
    pb, pw = t_ref.shape
    latent = pw // PACK
    gvec = jnp.dot(ga_ref[...], w1g_ref[...], preferred_element_type=jnp.float32)
    gvec = gvec + b1_ref[...]
    gvec = jnp.concatenate([gvec] * PACK, axis=1)
    pre = (t_ref[...]
           + jnp.dot(ea_ref[...], w1e_ref[...], preferred_element_type=jnp.float32)
           + gvec)
    h = jnp.maximum(pre, 0.0)
    for u in range(PACK):
        h_u = h[:, u * latent:(u + 1) * latent]
        o_u = jnp.dot(h_u, w2_ref[...], preferred_element_type=jnp.float32) + b2_ref[...]
        out_ref[pl.Slice(u, pb, PACK), :] = o_u


def kernel(node_attr, edge_attr, global_attr, edge_index, ng_index, eg_index,
           W1, b1, W2, b2):
    n_nodes, d_feat = node_attr.shape
    n_edges, d_edge = edge_attr.shape
    d_global = global_attr.shape[1]
    latent = W1.shape[1]
    out_dim = W2.shape[1]

    w1_e = W1[:d_edge]
    w1_s = W1[d_edge:d_edge + d_feat]
    w1_r = W1[d_edge + d_feat:d_edge + 2 * d_feat]
    w1_g = W1[d_edge + 2 * d_feat:]

    half = latent // 2
    perm = np.empty(latent, dtype=np.int32)
    perm[0::2] = np.arange(half)
    perm[1::2] = np.arange(half) + half

    proj_s, proj_r = pl.pallas_call(
        _node_proj_kernel,
        out_shape=(
            jax.ShapeDtypeStruct((n_nodes, latent), jnp.bfloat16),
            jax.ShapeDtypeStruct((n_nodes, latent), jnp.bfloat16),
        ),
    )(node_attr, w1_s[:, perm], w1_r[:, perm])

    per_worker_unit = NUM_WORKERS * CHUNK
    e_pad = ((n_edges + per_worker_unit - 1) // per_worker_unit) * per_worker_unit
    chunks_per_worker = e_pad // per_worker_unit

    s_idx = edge_index[0].astype(jnp.int32)
    r_idx = edge_index[1].astype(jnp.int32)
    pad = e_pad - n_edges
    s_idx = jnp.pad(s_idx, (0, pad)).reshape(e_pad // IDX_PER_STREAM, IDX_PER_STREAM)
    r_idx = jnp.pad(r_idx, (0, pad)).reshape(e_pad // IDX_PER_STREAM, IDX_PER_STREAM)

    t_packed = _make_sc_gather(e_pad, latent, chunks_per_worker)(
        s_idx, r_idx, proj_s, proj_r)

    eb = 8000
    grid = n_edges // eb
    pb = eb // PACK
    ea_packed = edge_attr.reshape(n_edges // PACK, PACK * d_edge)
    w1e_bd = jnp.kron(jnp.eye(PACK, dtype=jnp.float32), w1_e)
    out = pl.pallas_call(
        _edge_mlp_kernel,
        grid=(grid,),
        in_specs=[
            pl.BlockSpec((pb, PACK * d_edge), lambda i: (i, 0)),
            pl.BlockSpec((pb, PACK * latent), lambda i: (i, 0)),
            pl.BlockSpec((1, d_global), lambda i: (0, 0)),
            pl.BlockSpec((PACK * d_edge, PACK * latent), lambda i: (0, 0)),
            pl.BlockSpec((d_global, latent), lambda i: (0, 0)),
            pl.BlockSpec((1, latent), lambda i: (0, 0)),
            pl.BlockSpec((latent, out_dim), lambda i: (0, 0)),
            pl.BlockSpec((1, out_dim), lambda i: (0, 0)),
        ],
        out_specs=pl.BlockSpec((eb, out_dim), lambda i: (i, 0)),
        out_shape=jax.ShapeDtypeStruct((n_edges, out_dim), jnp.float32),
    )(ea_packed, t_packed, global_attr, w1e_bd, w1_g,
      b1.reshape(1, latent), W2, b2.reshape(1, out_dim))
    return out


# speedup vs baseline: 1.4171x; 1.0221x over previous
import functools

import jax
import jax.numpy as jnp
import numpy as np
from jax import lax
from jax.experimental import pallas as pl
from jax.experimental.pallas import tpu as pltpu
from jax.experimental.pallas import tpu_sc as plsc

NUM_CORES = 2
NUM_SUBCORES = 16
NUM_WORKERS = NUM_CORES * NUM_SUBCORES
IDX_PER_STREAM = 128
STREAMS_PER_CHUNK = 8
CHUNK = IDX_PER_STREAM * STREAMS_PER_CHUNK
PACK = 4
LANES = 16


def _node_proj_kernel(na_ref, w1s_ref, w1r_ref, ps_ref, pr_ref):
    na = na_ref[...]
    ps_ref[...] = jnp.dot(
        na, w1s_ref[...], preferred_element_type=jnp.float32).astype(jnp.bfloat16)
    pr_ref[...] = jnp.dot(
        na, w1r_ref[...], preferred_element_type=jnp.float32).astype(jnp.bfloat16)


def _make_sc_gather(e_pad, latent, chunks_per_worker):
    rows_per_chunk = STREAMS_PER_CHUNK
    packed_per_chunk = CHUNK // PACK
    packed_width = PACK * latent
    n_chunks = chunks_per_worker
    mesh = plsc.VectorSubcoreMesh(core_axis_name="c", subcore_axis_name="s")

    @functools.partial(
        pl.kernel,
        out_type=jax.ShapeDtypeStruct((e_pad // PACK, packed_width), jnp.float32),
        mesh=mesh,
        compiler_params=pltpu.CompilerParams(
            use_tc_tiling_on_sc=False, needs_layout_passes=False),
        scratch_types=[
            pltpu.VMEM((2, rows_per_chunk, IDX_PER_STREAM), jnp.int32),
            pltpu.VMEM((2, rows_per_chunk, IDX_PER_STREAM), jnp.int32),
            pltpu.VMEM((2 * CHUNK, latent), jnp.bfloat16),
            pltpu.VMEM((2 * CHUNK, latent), jnp.bfloat16),
            pltpu.VMEM((packed_per_chunk, packed_width), jnp.float32),
            pltpu.SemaphoreType.DMA,
            pltpu.SemaphoreType.DMA,
        ],
    )
    def sc_gather(sidx_hbm, ridx_hbm, ps_hbm, pr_hbm, t_hbm,
                  idxs_v, idxr_v, bufs_v, bufr_v, buft_v, sem0, sem1):
        c = lax.axis_index("c")
        s = lax.axis_index("s")
        wid = s * NUM_CORES + c
        chunk_base = wid * n_chunks
        sems = (sem0, sem1)

        def fire(g, b):
            idx_row = (chunk_base + g) * rows_per_chunk
            pltpu.sync_copy(sidx_hbm.at[pl.ds(idx_row, rows_per_chunk)],
                            idxs_v.at[b])
            pltpu.sync_copy(ridx_hbm.at[pl.ds(idx_row, rows_per_chunk)],
                            idxr_v.at[b])
            for j in range(STREAMS_PER_CHUNK):
                dst = pl.ds(b * CHUNK + j * IDX_PER_STREAM, IDX_PER_STREAM)
                pltpu.async_copy(ps_hbm.at[idxs_v.at[b, j]], bufs_v.at[dst], sems[b])
                pltpu.async_copy(pr_hbm.at[idxr_v.at[b, j]], bufr_v.at[dst], sems[b])

        def drain(b):
            for j in range(STREAMS_PER_CHUNK):
                dst = pl.ds(b * CHUNK + j * IDX_PER_STREAM, IDX_PER_STREAM)
                pltpu.make_async_copy(
                    ps_hbm.at[idxs_v.at[b, j]], bufs_v.at[dst], sems[b]).wait()
                pltpu.make_async_copy(
                    pr_hbm.at[idxr_v.at[b, j]], bufr_v.at[dst], sems[b]).wait()

        def pack_store(g, b):
            base = b * CHUNK

            def pack_body(q, c2):
                i0 = base + q * PACK
                for u in range(PACK):
                    i = i0 + u
                    sa, sb = plsc.unpack(bufs_v[i, :],
                                         format=plsc.PackFormat.INTERLEAVED)
                    ra, rb = plsc.unpack(bufr_v[i, :],
                                         format=plsc.PackFormat.INTERLEAVED)
                    col0 = u * latent
                    buft_v[q, pl.ds(col0, LANES)] = sa + ra
                    buft_v[q, pl.ds(col0 + LANES, LANES)] = sb + rb
                return c2

            lax.fori_loop(0, packed_per_chunk, pack_body, 0)
            out_base = (chunk_base + g) * packed_per_chunk
            pltpu.sync_copy(buft_v, t_hbm.at[pl.ds(out_base, packed_per_chunk)])

        fire(0, 0)

        def pair_body(p, carry):
            g0 = p * 2
            drain(0)
            fire(g0 + 1, 1)
            pack_store(g0, 0)
            drain(1)

            @pl.when(p < n_chunks // 2 - 1)
            def _():
                fire(g0 + 2, 0)

            pack_store(g0 + 1, 1)
            return carry

        lax.fori_loop(0, n_chunks // 2, pair_body, 0)

    return sc_gather


def _edge_mlp_kernel(ea_ref, t_ref, ga_ref, w1e_ref, w1g_ref, b1_ref,
                     w2_ref, b2_ref, out_ref):
    pb, pw = t_ref.shape
    latent = pw // PACK
    gvec = jnp.dot(ga_ref[...], w1g_ref[...], preferred_element_type=jnp.float32)
    gvec = gvec + b1_ref[...]
    gvec = jnp.concatenate([gvec] * PACK, axis=1)
    pre = (t_ref[...]
           + jnp.dot(ea_ref[...], w1e_ref[...], preferred_element_type=jnp.float32)
           + gvec)
    h = jnp.maximum(pre, 0.0)
    for u in range(PACK):
        h_u = h[:, u * latent:(u + 1) * latent]
        o_u = jnp.dot(h_u, w2_ref[...], preferred_element_type=jnp.float32) + b2_ref[...]
        out_ref[pl.Slice(u, pb, PACK), :] = o_u


def kernel(node_attr, edge_attr, global_attr, edge_index, ng_index, eg_index,
           W1, b1, W2, b2):
    n_nodes, d_feat = node_attr.shape
    n_edges, d_edge = edge_attr.shape
    d_global = global_attr.shape[1]
    latent = W1.shape[1]
    out_dim = W2.shape[1]

    w1_e = W1[:d_edge]
    w1_s = W1[d_edge:d_edge + d_feat]
    w1_r = W1[d_edge + d_feat:d_edge + 2 * d_feat]
    w1_g = W1[d_edge + 2 * d_feat:]

    half = latent // 2
    perm = np.empty(latent, dtype=np.int32)
    perm[0::2] = np.arange(half)
    perm[1::2] = np.arange(half) + half

    proj_s, proj_r = pl.pallas_call(
        _node_proj_kernel,
        out_shape=(
            jax.ShapeDtypeStruct((n_nodes, latent), jnp.bfloat16),
            jax.ShapeDtypeStruct((n_nodes, latent), jnp.bfloat16),
        ),
    )(node_attr, w1_s[:, perm], w1_r[:, perm])

    per_worker_unit = NUM_WORKERS * CHUNK
    e_pad = ((n_edges + per_worker_unit - 1) // per_worker_unit) * per_worker_unit
    chunks_per_worker = e_pad // per_worker_unit

    s_idx = edge_index[0].astype(jnp.int32)
    r_idx = edge_index[1].astype(jnp.int32)
    pad = e_pad - n_edges
    s_idx = jnp.pad(s_idx, (0, pad)).reshape(e_pad // IDX_PER_STREAM, IDX_PER_STREAM)
    r_idx = jnp.pad(r_idx, (0, pad)).reshape(e_pad // IDX_PER_STREAM, IDX_PER_STREAM)

    t_packed = _make_sc_gather(e_pad, latent, chunks_per_worker)(
        s_idx, r_idx, proj_s, proj_r)

    eb = 16000
    grid = n_edges // eb
    pb = eb // PACK
    ea_packed = edge_attr.reshape(n_edges // PACK, PACK * d_edge)
    w1e_bd = jnp.kron(jnp.eye(PACK, dtype=jnp.float32), w1_e)
    out = pl.pallas_call(
        _edge_mlp_kernel,
        grid=(grid,),
        in_specs=[
            pl.BlockSpec((pb, PACK * d_edge), lambda i: (i, 0)),
            pl.BlockSpec((pb, PACK * latent), lambda i: (i, 0)),
            pl.BlockSpec((1, d_global), lambda i: (0, 0)),
            pl.BlockSpec((PACK * d_edge, PACK * latent), lambda i: (0, 0)),
            pl.BlockSpec((d_global, latent), lambda i: (0, 0)),
            pl.BlockSpec((1, latent), lambda i: (0, 0)),
            pl.BlockSpec((latent, out_dim), lambda i: (0, 0)),
            pl.BlockSpec((1, out_dim), lambda i: (0, 0)),
        ],
        out_specs=pl.BlockSpec((eb, out_dim), lambda i: (i, 0)),
        out_shape=jax.ShapeDtypeStruct((n_edges, out_dim), jnp.float32),
    )(ea_packed, t_packed, global_attr, w1e_bd, w1_g,
      b1.reshape(1, latent), W2, b2.reshape(1, out_dim))
    return out


# speedup vs baseline: 1.4277x; 1.0075x over previous
import functools

import jax
import jax.numpy as jnp
import numpy as np
from jax import lax
from jax.experimental import pallas as pl
from jax.experimental.pallas import tpu as pltpu
from jax.experimental.pallas import tpu_sc as plsc

NUM_CORES = 2
NUM_SUBCORES = 16
NUM_WORKERS = NUM_CORES * NUM_SUBCORES
IDX_PER_STREAM = 128
STREAMS_PER_CHUNK = 8
CHUNK = IDX_PER_STREAM * STREAMS_PER_CHUNK
PACK = 4
LANES = 16


def _node_proj_kernel(na_ref, w1s_ref, w1r_ref, ps_ref, pr_ref):
    na = na_ref[...]
    ps_ref[...] = jnp.dot(
        na, w1s_ref[...], preferred_element_type=jnp.float32).astype(jnp.bfloat16)
    pr_ref[...] = jnp.dot(
        na, w1r_ref[...], preferred_element_type=jnp.float32).astype(jnp.bfloat16)


def _make_sc_gather(e_pad, latent, chunks_per_worker):
    rows_per_chunk = STREAMS_PER_CHUNK
    packed_per_chunk = CHUNK // PACK
    packed_width = PACK * latent
    n_chunks = chunks_per_worker
    mesh = plsc.VectorSubcoreMesh(core_axis_name="c", subcore_axis_name="s")

    @functools.partial(
        pl.kernel,
        out_type=jax.ShapeDtypeStruct((e_pad // PACK, packed_width), jnp.float32),
        mesh=mesh,
        compiler_params=pltpu.CompilerParams(
            use_tc_tiling_on_sc=False, needs_layout_passes=False),
        scratch_types=[
            pltpu.VMEM((2, rows_per_chunk, IDX_PER_STREAM), jnp.int32),
            pltpu.VMEM((2, rows_per_chunk, IDX_PER_STREAM), jnp.int32),
            pltpu.VMEM((2 * CHUNK, latent), jnp.bfloat16),
            pltpu.VMEM((2 * CHUNK, latent), jnp.bfloat16),
            pltpu.VMEM((packed_per_chunk, packed_width), jnp.float32),
            pltpu.SemaphoreType.DMA,
            pltpu.SemaphoreType.DMA,
        ],
    )
    def sc_gather(sidx_hbm, ridx_hbm, ps_hbm, pr_hbm, t_hbm,
                  idxs_v, idxr_v, bufs_v, bufr_v, buft_v, sem0, sem1):
        c = lax.axis_index("c")
        s = lax.axis_index("s")
        wid = s * NUM_CORES + c
        chunk_base = wid * n_chunks
        sems = (sem0, sem1)

        def fire(g, b):
            idx_row = (chunk_base + g) * rows_per_chunk
            pltpu.sync_copy(sidx_hbm.at[pl.ds(idx_row, rows_per_chunk)],
                            idxs_v.at[b])
            pltpu.sync_copy(ridx_hbm.at[pl.ds(idx_row, rows_per_chunk)],
                            idxr_v.at[b])
            for j in range(STREAMS_PER_CHUNK):
                dst = pl.ds(b * CHUNK + j * IDX_PER_STREAM, IDX_PER_STREAM)
                pltpu.async_copy(ps_hbm.at[idxs_v.at[b, j]], bufs_v.at[dst], sems[b])
                pltpu.async_copy(pr_hbm.at[idxr_v.at[b, j]], bufr_v.at[dst], sems[b])

        def drain(b):
            for j in range(STREAMS_PER_CHUNK):
                dst = pl.ds(b * CHUNK + j * IDX_PER_STREAM, IDX_PER_STREAM)
                pltpu.make_async_copy(
                    ps_hbm.at[idxs_v.at[b, j]], bufs_v.at[dst], sems[b]).wait()
                pltpu.make_async_copy(
                    pr_hbm.at[idxr_v.at[b, j]], bufr_v.at[dst], sems[b]).wait()

        def pack_store(g, b):
            base = b * CHUNK

            def pack_body(q, c2):
                i0 = base + q * PACK
                for u in range(PACK):
                    i = i0 + u
                    sa, sb = plsc.unpack(bufs_v[i, :],
                                         format=plsc.PackFormat.INTERLEAVED)
                    ra, rb = plsc.unpack(bufr_v[i, :],
                                         format=plsc.PackFormat.INTERLEAVED)
                    col0 = u * latent
                    buft_v[q, pl.ds(col0, LANES)] = sa + ra
                    buft_v[q, pl.ds(col0 + LANES, LANES)] = sb + rb
                return c2

            lax.fori_loop(0, packed_per_chunk, pack_body, 0)
            out_base = (chunk_base + g) * packed_per_chunk
            pltpu.sync_copy(buft_v, t_hbm.at[pl.ds(out_base, packed_per_chunk)])

        fire(0, 0)

        def pair_body(p, carry):
            g0 = p * 2
            drain(0)
            fire(g0 + 1, 1)
            pack_store(g0, 0)
            drain(1)

            @pl.when(p < n_chunks // 2 - 1)
            def _():
                fire(g0 + 2, 0)

            pack_store(g0 + 1, 1)
            return carry

        lax.fori_loop(0, n_chunks // 2, pair_body, 0)

    return sc_gather


def _edge_mlp_kernel(ea_ref, t_ref, ga_ref, w1e_ref, w1g_ref, b1_ref,
                     w2_ref, b2_ref, out_ref):
    pb, pw = t_ref.shape
    latent = pw // PACK
    gvec = jnp.dot(ga_ref[...], w1g_ref[...], preferred_element_type=jnp.float32)
    gvec = gvec + b1_ref[...]
    gvec = jnp.concatenate([gvec] * PACK, axis=1)
    pre = (t_ref[...]
           + jnp.dot(ea_ref[...], w1e_ref[...], preferred_element_type=jnp.float32)
           + gvec)
    h = jnp.maximum(pre, 0.0)
    for u in range(PACK):
        h_u = h[:, u * latent:(u + 1) * latent]
        o_u = jnp.dot(h_u, w2_ref[...], preferred_element_type=jnp.float32) + b2_ref[...]
        out_ref[pl.Slice(u, pb, PACK), :] = o_u


def kernel(node_attr, edge_attr, global_attr, edge_index, ng_index, eg_index,
           W1, b1, W2, b2):
    n_nodes, d_feat = node_attr.shape
    n_edges, d_edge = edge_attr.shape
    d_global = global_attr.shape[1]
    latent = W1.shape[1]
    out_dim = W2.shape[1]

    w1_e = W1[:d_edge]
    w1_s = W1[d_edge:d_edge + d_feat]
    w1_r = W1[d_edge + d_feat:d_edge + 2 * d_feat]
    w1_g = W1[d_edge + 2 * d_feat:]

    half = latent // 2
    perm = np.empty(latent, dtype=np.int32)
    perm[0::2] = np.arange(half)
    perm[1::2] = np.arange(half) + half

    proj_s, proj_r = pl.pallas_call(
        _node_proj_kernel,
        out_shape=(
            jax.ShapeDtypeStruct((n_nodes, latent), jnp.bfloat16),
            jax.ShapeDtypeStruct((n_nodes, latent), jnp.bfloat16),
        ),
    )(node_attr, w1_s[:, perm], w1_r[:, perm])

    per_worker_unit = NUM_WORKERS * CHUNK
    e_pad = ((n_edges + per_worker_unit - 1) // per_worker_unit) * per_worker_unit
    chunks_per_worker = e_pad // per_worker_unit

    s_idx = edge_index[0].astype(jnp.int32)
    r_idx = edge_index[1].astype(jnp.int32)
    pad = e_pad - n_edges
    s_idx = jnp.pad(s_idx, (0, pad)).reshape(e_pad // IDX_PER_STREAM, IDX_PER_STREAM)
    r_idx = jnp.pad(r_idx, (0, pad)).reshape(e_pad // IDX_PER_STREAM, IDX_PER_STREAM)

    t_packed = _make_sc_gather(e_pad, latent, chunks_per_worker)(
        s_idx, r_idx, proj_s, proj_r)

    eb = 32000
    grid = n_edges // eb
    pb = eb // PACK
    ea_packed = edge_attr.reshape(n_edges // PACK, PACK * d_edge)
    w1e_bd = jnp.kron(jnp.eye(PACK, dtype=jnp.float32), w1_e)
    out = pl.pallas_call(
        _edge_mlp_kernel,
        grid=(grid,),
        in_specs=[
            pl.BlockSpec((pb, PACK * d_edge), lambda i: (i, 0)),
            pl.BlockSpec((pb, PACK * latent), lambda i: (i, 0)),
            pl.BlockSpec((1, d_global), lambda i: (0, 0)),
            pl.BlockSpec((PACK * d_edge, PACK * latent), lambda i: (0, 0)),
            pl.BlockSpec((d_global, latent), lambda i: (0, 0)),
            pl.BlockSpec((1, latent), lambda i: (0, 0)),
            pl.BlockSpec((latent, out_dim), lambda i: (0, 0)),
            pl.BlockSpec((1, out_dim), lambda i: (0, 0)),
        ],
        out_specs=pl.BlockSpec((eb, out_dim), lambda i: (i, 0)),
        out_shape=jax.ShapeDtypeStruct((n_edges, out_dim), jnp.float32),
    )(ea_packed, t_packed, global_attr, w1e_bd, w1_g,
      b1.reshape(1, latent), W2, b2.reshape(1, out_dim))
    return out
